# baseline (speedup 1.0000x reference)
import jax
import jax.numpy as jnp
from jax import lax
from jax.experimental import pallas as pl
from jax.experimental.pallas import tpu as pltpu

_B = 2
_N = 5000
_NP = 5120
_MAX_GT = 100
_POS = 64
_NEG = 192
_TRAIN = 256
_NUM_CLASSES = 81
_MASK_IN = 128
_MASK_H = 28
_MASK_W = 28
_EPS = 1e-6
_INV_SCALE = 1.0 / 512.0
_HI = lax.Precision.HIGHEST


def _to8(row):
    return jnp.concatenate([row[:, 640 * r:640 * (r + 1)] for r in range(8)],
                           axis=0)


def _flat(t8):
    return jnp.concatenate([t8[r:r + 1, :] for r in range(8)], axis=1)


def _roi_kernel(props_ref, propsT_ref, gt_ref, gtT_ref, clsf_ref, masks_ref,
                rois_ref, cls_ref, deltas_ref, masks_out_ref,
                pr_ref, asg_ref, wy_ref, wx_ref):
    lane = lax.broadcasted_iota(jnp.int32, (1, _NP), 1)
    idx8 = (lax.broadcasted_iota(jnp.int32, (8, 640), 0) * 640
            + lax.broadcasted_iota(jnp.int32, (8, 640), 1))

    m_pos = []
    m_neg = []
    for b in range(_B):
        gt_n = gt_ref[b] * _INV_SCALE
        pT = propsT_ref[b]
        g0 = gt_n[:, 0:1]
        g1 = gt_n[:, 1:2]
        g2 = gt_n[:, 2:3]
        g3 = gt_n[:, 3:4]
        p0 = pT[0:1, :]
        p1 = pT[1:2, :]
        p2 = pT[2:3, :]
        p3 = pT[3:4, :]
        yy1 = jnp.maximum(g0, p0)
        xx1 = jnp.maximum(g1, p1)
        yy2 = jnp.minimum(g2, p2)
        xx2 = jnp.minimum(g3, p3)
        inter = jnp.maximum(yy2 - yy1, 0.0) * jnp.maximum(xx2 - xx1, 0.0)
        area_p = (p2 - p0) * (p3 - p1)
        area_g = (g2 - g0) * (g3 - g1)
        ovT = inter / (area_p + area_g - inter + _EPS)
        rmax = jnp.max(ovT, axis=0, keepdims=True)
        valid = lane < _N
        m_pos.append(_to8(jnp.where(valid, rmax, -1.0)))
        m_neg.append(_to8(jnp.where(valid, rmax, 2.0)))

    def _extract_max(m, stamp):
        v = jnp.max(jnp.max(m, axis=1, keepdims=True), axis=0,
                    keepdims=True)
        t = jnp.where(m == v, idx8, _NP)
        j = jnp.min(jnp.min(t, axis=1, keepdims=True), axis=0,
                    keepdims=True)
        return jnp.where(idx8 == j, stamp, m)

    def _extract_min(n, stamp):
        w = jnp.min(jnp.min(n, axis=1, keepdims=True), axis=0,
                    keepdims=True)
        t = jnp.where(n == w, idx8, -1)
        k = jnp.max(jnp.max(t, axis=1, keepdims=True), axis=0,
                    keepdims=True)
        return jnp.where(idx8 == k, stamp, n)

    def both_body(i, c):
        m0, m1, n0, n1 = c
        fi = i.astype(jnp.float32)
        pstamp = -3.0 - fi
        nstamp = 258.0 - fi
        m0 = _extract_max(m0, pstamp)
        m1 = _extract_max(m1, pstamp)
        n0 = _extract_min(n0, nstamp)
        n1 = _extract_min(n1, nstamp)
        return m0, m1, n0, n1

    m0, m1, n0, n1 = lax.fori_loop(
        0, _POS, both_body, (m_pos[0], m_pos[1], m_neg[0], m_neg[1]))

    def neg_body(i, c):
        n0, n1 = c
        nstamp = 258.0 - i.astype(jnp.float32)
        n0 = _extract_min(n0, nstamp)
        n1 = _extract_min(n1, nstamp)
        return n0, n1

    n0, n1 = lax.fori_loop(_POS, _NEG, neg_body, (n0, n1))

    rposf = lax.broadcasted_iota(jnp.int32, (_POS, 1), 0).astype(
        jnp.float32) + 1.0
    rnegf = lax.broadcasted_iota(jnp.int32, (_NEG, 1), 0).astype(
        jnp.float32) + (_POS + 1.0)
    i100 = lax.broadcasted_iota(jnp.int32, (_POS, _MAX_GT), 1)
    lane324 = lax.broadcasted_iota(jnp.int32, (_POS, _NUM_CLASSES * 4), 1)

    for b, msel, nsel in ((0, m0, n0), (1, m1, n1)):
        pm_row = -_flat(msel) - 2.0
        nm_row = _flat(nsel) - 2.0
        P = (pm_row == rposf).astype(jnp.float32)
        Nh = (nm_row == rnegf).astype(jnp.float32)
        props = props_ref[b]
        pos_rois = jnp.dot(P, props, preferred_element_type=jnp.float32,
                           precision=_HI)
        neg_rois = jnp.dot(Nh, props, preferred_element_type=jnp.float32,
                           precision=_HI)

        gtT_n = gtT_ref[b] * _INV_SCALE
        q0 = gtT_n[0:1, :]
        q1 = gtT_n[1:2, :]
        q2 = gtT_n[2:3, :]
        q3 = gtT_n[3:4, :]
        r0 = pos_rois[:, 0:1]
        r1 = pos_rois[:, 1:2]
        r2 = pos_rois[:, 2:3]
        r3 = pos_rois[:, 3:4]
        py1 = jnp.maximum(r0, q0)
        px1 = jnp.maximum(r1, q1)
        py2 = jnp.minimum(r2, q2)
        px2 = jnp.minimum(r3, q3)
        pinter = jnp.maximum(py2 - py1, 0.0) * jnp.maximum(px2 - px1, 0.0)
        par = (r2 - r0) * (r3 - r1)
        pag = (q2 - q0) * (q3 - q1)
        po = pinter / (par + pag - pinter + _EPS)

        amax = jnp.max(po, axis=1, keepdims=True)
        asg = jnp.min(jnp.where(po == amax, i100, _MAX_GT), axis=1,
                      keepdims=True)
        A = (i100 == asg).astype(jnp.float32)
        gt_n = gt_ref[b] * _INV_SCALE
        roi_gt = jnp.dot(A, gt_n, preferred_element_type=jnp.float32,
                         precision=_HI)
        idsf = jnp.dot(A, clsf_ref[b], preferred_element_type=jnp.float32,
                       precision=_HI)
        ids = (idsf + 0.5).astype(jnp.int32)

        h = pos_rois[:, 2:3] - pos_rois[:, 0:1] + _EPS
        w = pos_rois[:, 3:4] - pos_rois[:, 1:2] + _EPS
        cy = pos_rois[:, 0:1] + 0.5 * h
        cx = pos_rois[:, 1:2] + 0.5 * w
        gh = roi_gt[:, 2:3] - roi_gt[:, 0:1] + _EPS
        gw = roi_gt[:, 3:4] - roi_gt[:, 1:2] + _EPS
        gcy = roi_gt[:, 0:1] + 0.5 * gh
        gcx = roi_gt[:, 1:2] + 0.5 * gw
        dy = ((gcy - cy) / h) / 0.1
        dx = ((gcx - cx) / w) / 0.1
        dh = jnp.log(gh / h) / 0.2
        dw = jnp.log(gw / w) / 0.2

        cls_l = lane324 // 4
        d_l = lane324 % 4
        dval = jnp.where(d_l == 0, dy,
                         jnp.where(d_l == 1, dx,
                                   jnp.where(d_l == 2, dh, dw)))
        cd = jnp.where(cls_l == ids, dval, 0.0)

        lin28c = lax.broadcasted_iota(jnp.int32, (_POS, _MASK_H), 1).astype(
            jnp.float32) * (1.0 / (_MASK_H - 1))
        iota3 = lax.broadcasted_iota(jnp.int32,
                                     (_POS, _MASK_H, _MASK_IN), 2)
        ys = jnp.clip((r0 + (r2 - r0) * lin28c) * (_MASK_IN - 1.0),
                      0.0, _MASK_IN - 1.0)
        y0f = jnp.floor(ys)
        wy = (ys - y0f)[:, :, None]
        y0i = y0f.astype(jnp.int32)[:, :, None]
        y1i = jnp.minimum(y0i + 1, _MASK_IN - 1)
        wy_ref[pl.ds(_POS * b, _POS)] = (
            jnp.where(iota3 == y0i, 1.0 - wy, 0.0)
            + jnp.where(iota3 == y1i, wy, 0.0))
        xs = jnp.clip((r1 + (r3 - r1) * lin28c) * (_MASK_IN - 1.0),
                      0.0, _MASK_IN - 1.0)
        x0f = jnp.floor(xs)
        wx = (xs - x0f)[:, :, None]
        x0i = x0f.astype(jnp.int32)[:, :, None]
        x1i = jnp.minimum(x0i + 1, _MASK_IN - 1)
        wx_ref[pl.ds(_POS * b, _POS)] = (
            jnp.where(iota3 == x0i, 1.0 - wx, 0.0)
            + jnp.where(iota3 == x1i, wx, 0.0))

        pr_ref[pl.ds(_POS * b, _POS), :] = pos_rois
        asg_ref[pl.ds(_POS * b, _POS), :] = asg
        rois_ref[b, pl.ds(0, _POS), :] = pos_rois
        rois_ref[b, pl.ds(_POS, _NEG), :] = neg_rois
        cls_ref[b, pl.ds(0, _POS), :] = ids
        cls_ref[b, pl.ds(_POS, _NEG), :] = jnp.zeros((_NEG, 1), jnp.int32)
        deltas_ref[b, pl.ds(0, _POS), :] = cd
        deltas_ref[b, pl.ds(_POS, _NEG), :] = jnp.zeros(
            (_NEG, _NUM_CLASSES * 4), jnp.float32)
        masks_out_ref[b, pl.ds(_POS, _NEG)] = jnp.zeros(
            (_NEG, _MASK_H, _MASK_W), jnp.float32)

    def mask_body(k, carry):
        for u in range(4):
            r = k + 32 * u
            b = u // 2
            rr = k + 32 * (u % 2)
            ch = asg_ref[r, 0]
            M = masks_ref[b, ch]
            Wy = wy_ref[r]
            Wx = wx_ref[r]
            tmp = jnp.dot(Wy, M, preferred_element_type=jnp.float32,
                          precision=_HI)
            out = lax.dot_general(
                tmp, Wx, (((1,), (1,)), ((), ())),
                preferred_element_type=jnp.float32,
                precision=_HI)
            masks_out_ref[b, rr] = out
        return carry

    lax.fori_loop(0, 32, mask_body, 0)


def kernel(proposals, gt_class_ids, gt_boxes, gt_masks):
    props_pad = jnp.pad(proposals, ((0, 0), (0, _NP - _N), (0, 0)))
    propsT = jnp.transpose(props_pad, (0, 2, 1))
    gtT = jnp.transpose(gt_boxes, (0, 2, 1))
    clsf = gt_class_ids.astype(jnp.float32)[..., None]
    masksT = jnp.transpose(gt_masks, (0, 3, 1, 2))

    rois, cls3, deltas2, masks = pl.pallas_call(
        _roi_kernel,
        out_shape=[
            jax.ShapeDtypeStruct((_B, _TRAIN, 4), jnp.float32),
            jax.ShapeDtypeStruct((_B, _TRAIN, 1), jnp.int32),
            jax.ShapeDtypeStruct((_B, _TRAIN, _NUM_CLASSES * 4), jnp.float32),
            jax.ShapeDtypeStruct((_B, _TRAIN, _MASK_H, _MASK_W), jnp.float32),
        ],
        scratch_shapes=[
            pltpu.VMEM((_B * _POS, 4), jnp.float32),
            pltpu.VMEM((_B * _POS, 1), jnp.int32),
            pltpu.VMEM((_B * _POS, _MASK_H, _MASK_IN), jnp.float32),
            pltpu.VMEM((_B * _POS, _MASK_H, _MASK_IN), jnp.float32),
        ],
    )(props_pad, propsT, gt_boxes, gtT, clsf, masksT)

    cls = cls3[..., 0]
    deltas = deltas2.reshape(_B, _TRAIN, _NUM_CLASSES, 4)
    return rois, cls, deltas, masks


# speedup vs baseline: 1.1704x; 1.1704x over previous
import jax
import jax.numpy as jnp
from jax import lax
from jax.experimental import pallas as pl
from jax.experimental.pallas import tpu as pltpu

_B = 2
_N = 5000
_NP = 5120
_MAX_GT = 100
_POS = 64
_NEG = 192
_TRAIN = 256
_NUM_CLASSES = 81
_MASK_IN = 128
_MASK_H = 28
_MASK_W = 28
_EPS = 1e-6
_INV_SCALE = 1.0 / 512.0
_HI = lax.Precision.HIGHEST


def _to8(row):
    return jnp.concatenate([row[:, 640 * r:640 * (r + 1)] for r in range(8)],
                           axis=0)


def _flat(t8):
    return jnp.concatenate([t8[r:r + 1, :] for r in range(8)], axis=1)


def _roi_kernel(props_ref, propsT_ref, gt_ref, gtT_ref, clsf_ref, masks_ref,
                rois_ref, cls_ref, deltas_ref, masks_out_ref,
                pr_ref, asg_ref, wy_ref):
    lane = lax.broadcasted_iota(jnp.int32, (1, _NP), 1)
    idx8 = (lax.broadcasted_iota(jnp.int32, (8, 640), 0) * 640
            + lax.broadcasted_iota(jnp.int32, (8, 640), 1))

    m_pos = []
    m_neg = []
    for b in range(_B):
        gt_n = gt_ref[b] * _INV_SCALE
        pT = propsT_ref[b]
        g0 = gt_n[:, 0:1]
        g1 = gt_n[:, 1:2]
        g2 = gt_n[:, 2:3]
        g3 = gt_n[:, 3:4]
        p0 = pT[0:1, :]
        p1 = pT[1:2, :]
        p2 = pT[2:3, :]
        p3 = pT[3:4, :]
        yy1 = jnp.maximum(g0, p0)
        xx1 = jnp.maximum(g1, p1)
        yy2 = jnp.minimum(g2, p2)
        xx2 = jnp.minimum(g3, p3)
        inter = jnp.maximum(yy2 - yy1, 0.0) * jnp.maximum(xx2 - xx1, 0.0)
        area_p = (p2 - p0) * (p3 - p1)
        area_g = (g2 - g0) * (g3 - g1)
        ovT = inter / (area_p + area_g - inter + _EPS)
        rmax = jnp.max(ovT, axis=0, keepdims=True)
        valid = lane < _N
        m_pos.append(_to8(jnp.where(valid, rmax, -1.0)))
        m_neg.append(_to8(jnp.where(valid, rmax, 2.0)))

    def _extract_max(m, stamp):
        v = jnp.max(jnp.max(m, axis=1, keepdims=True), axis=0,
                    keepdims=True)
        t = jnp.where(m == v, idx8, _NP)
        j = jnp.min(jnp.min(t, axis=1, keepdims=True), axis=0,
                    keepdims=True)
        return jnp.where(idx8 == j, stamp, m)

    def _extract_min(n, stamp):
        w = jnp.min(jnp.min(n, axis=1, keepdims=True), axis=0,
                    keepdims=True)
        t = jnp.where(n == w, idx8, -1)
        k = jnp.max(jnp.max(t, axis=1, keepdims=True), axis=0,
                    keepdims=True)
        return jnp.where(idx8 == k, stamp, n)

    def both_body(i, c):
        m0, m1, n0, n1 = c
        fi = i.astype(jnp.float32)
        pstamp = -3.0 - fi
        nstamp = 258.0 - fi
        m0 = _extract_max(m0, pstamp)
        m1 = _extract_max(m1, pstamp)
        n0 = _extract_min(n0, nstamp)
        n1 = _extract_min(n1, nstamp)
        return m0, m1, n0, n1

    m0, m1, n0, n1 = lax.fori_loop(
        0, _POS, both_body, (m_pos[0], m_pos[1], m_neg[0], m_neg[1]))

    def neg_body(i, c):
        n0, n1 = c
        nstamp = 258.0 - i.astype(jnp.float32)
        n0 = _extract_min(n0, nstamp)
        n1 = _extract_min(n1, nstamp)
        return n0, n1

    n0, n1 = lax.fori_loop(_POS, _NEG, neg_body, (n0, n1))

    rposf = lax.broadcasted_iota(jnp.int32, (_POS, 1), 0).astype(
        jnp.float32) + 1.0
    rnegf = lax.broadcasted_iota(jnp.int32, (_NEG, 1), 0).astype(
        jnp.float32) + (_POS + 1.0)
    i100 = lax.broadcasted_iota(jnp.int32, (_POS, _MAX_GT), 1)
    lane324 = lax.broadcasted_iota(jnp.int32, (_POS, _NUM_CLASSES * 4), 1)

    for b, msel, nsel in ((0, m0, n0), (1, m1, n1)):
        pm_row = -_flat(msel) - 2.0
        nm_row = _flat(nsel) - 2.0
        P = (pm_row == rposf).astype(jnp.float32)
        Nh = (nm_row == rnegf).astype(jnp.float32)
        props = props_ref[b]
        pos_rois = jnp.dot(P, props, preferred_element_type=jnp.float32,
                           precision=_HI)
        neg_rois = jnp.dot(Nh, props, preferred_element_type=jnp.float32,
                           precision=_HI)

        gtT_n = gtT_ref[b] * _INV_SCALE
        q0 = gtT_n[0:1, :]
        q1 = gtT_n[1:2, :]
        q2 = gtT_n[2:3, :]
        q3 = gtT_n[3:4, :]
        r0 = pos_rois[:, 0:1]
        r1 = pos_rois[:, 1:2]
        r2 = pos_rois[:, 2:3]
        r3 = pos_rois[:, 3:4]
        py1 = jnp.maximum(r0, q0)
        px1 = jnp.maximum(r1, q1)
        py2 = jnp.minimum(r2, q2)
        px2 = jnp.minimum(r3, q3)
        pinter = jnp.maximum(py2 - py1, 0.0) * jnp.maximum(px2 - px1, 0.0)
        par = (r2 - r0) * (r3 - r1)
        pag = (q2 - q0) * (q3 - q1)
        po = pinter / (par + pag - pinter + _EPS)

        amax = jnp.max(po, axis=1, keepdims=True)
        asg = jnp.min(jnp.where(po == amax, i100, _MAX_GT), axis=1,
                      keepdims=True)
        A = (i100 == asg).astype(jnp.float32)
        gt_n = gt_ref[b] * _INV_SCALE
        roi_gt = jnp.dot(A, gt_n, preferred_element_type=jnp.float32,
                         precision=_HI)
        idsf = jnp.dot(A, clsf_ref[b], preferred_element_type=jnp.float32,
                       precision=_HI)
        ids = (idsf + 0.5).astype(jnp.int32)

        h = pos_rois[:, 2:3] - pos_rois[:, 0:1] + _EPS
        w = pos_rois[:, 3:4] - pos_rois[:, 1:2] + _EPS
        cy = pos_rois[:, 0:1] + 0.5 * h
        cx = pos_rois[:, 1:2] + 0.5 * w
        gh = roi_gt[:, 2:3] - roi_gt[:, 0:1] + _EPS
        gw = roi_gt[:, 3:4] - roi_gt[:, 1:2] + _EPS
        gcy = roi_gt[:, 0:1] + 0.5 * gh
        gcx = roi_gt[:, 1:2] + 0.5 * gw
        dy = ((gcy - cy) / h) / 0.1
        dx = ((gcx - cx) / w) / 0.1
        dh = jnp.log(gh / h) / 0.2
        dw = jnp.log(gw / w) / 0.2

        cls_l = lane324 // 4
        d_l = lane324 % 4
        dval = jnp.where(d_l == 0, dy,
                         jnp.where(d_l == 1, dx,
                                   jnp.where(d_l == 2, dh, dw)))
        cd = jnp.where(cls_l == ids, dval, 0.0)

        lin28c = lax.broadcasted_iota(jnp.int32, (_POS, _MASK_H), 1).astype(
            jnp.float32) * (1.0 / (_MASK_H - 1))
        iota3 = lax.broadcasted_iota(jnp.int32,
                                     (_POS, _MASK_H, _MASK_IN), 2)
        ys = jnp.clip((r0 + (r2 - r0) * lin28c) * (_MASK_IN - 1.0),
                      0.0, _MASK_IN - 1.0)
        y0f = jnp.floor(ys)
        wy = (ys - y0f)[:, :, None]
        y0i = y0f.astype(jnp.int32)[:, :, None]
        y1i = jnp.minimum(y0i + 1, _MASK_IN - 1)
        wy_ref[pl.ds(_POS * b, _POS)] = (
            jnp.where(iota3 == y0i, 1.0 - wy, 0.0)
            + jnp.where(iota3 == y1i, wy, 0.0))
        pr_ref[pl.ds(_POS * b, _POS), :] = pos_rois
        asg_ref[pl.ds(_POS * b, _POS), :] = asg
        rois_ref[b, pl.ds(0, _POS), :] = pos_rois
        rois_ref[b, pl.ds(_POS, _NEG), :] = neg_rois
        cls_ref[b, pl.ds(0, _POS), :] = ids
        cls_ref[b, pl.ds(_POS, _NEG), :] = jnp.zeros((_NEG, 1), jnp.int32)
        deltas_ref[b, pl.ds(0, _POS), :] = cd
        deltas_ref[b, pl.ds(_POS, _NEG), :] = jnp.zeros(
            (_NEG, _NUM_CLASSES * 4), jnp.float32)
        masks_out_ref[b, pl.ds(_POS, _NEG)] = jnp.zeros(
            (_NEG, _MASK_H, _MASK_W), jnp.float32)

    lin_row = lax.broadcasted_iota(jnp.int32, (1, _MASK_W), 1).astype(
        jnp.float32) * (1.0 / (_MASK_W - 1))
    xj = lax.broadcasted_iota(jnp.int32, (_MASK_IN, _MASK_W), 0)

    def mask_body(k, carry):
        for u in range(8):
            r = k + 16 * u
            b = u // 4
            rr = k + 16 * (u % 4)
            row = pr_ref[pl.ds(r, 1), :]
            b1 = row[:, 1:2]
            b3 = row[:, 3:4]
            ch = asg_ref[r, 0]
            M = masks_ref[b, ch]
            Wy = wy_ref[r]

            xs = jnp.clip((b1 + (b3 - b1) * lin_row) * (_MASK_IN - 1.0),
                          0.0, _MASK_IN - 1.0)
            x0f = jnp.floor(xs)
            wx = xs - x0f
            x0i = x0f.astype(jnp.int32)
            x1i = jnp.minimum(x0i + 1, _MASK_IN - 1)
            WxT = (jnp.where(xj == x0i, 1.0 - wx, 0.0)
                   + jnp.where(xj == x1i, wx, 0.0))

            tmp = jnp.dot(Wy, M, preferred_element_type=jnp.float32,
                          precision=_HI)
            out = jnp.dot(tmp, WxT, preferred_element_type=jnp.float32,
                          precision=_HI)
            masks_out_ref[b, rr] = out
        return carry

    lax.fori_loop(0, 16, mask_body, 0)


def kernel(proposals, gt_class_ids, gt_boxes, gt_masks):
    props_pad = jnp.pad(proposals, ((0, 0), (0, _NP - _N), (0, 0)))
    propsT = jnp.transpose(props_pad, (0, 2, 1))
    gtT = jnp.transpose(gt_boxes, (0, 2, 1))
    clsf = gt_class_ids.astype(jnp.float32)[..., None]
    masksT = jnp.transpose(gt_masks, (0, 3, 1, 2))

    rois, cls3, deltas2, masks = pl.pallas_call(
        _roi_kernel,
        out_shape=[
            jax.ShapeDtypeStruct((_B, _TRAIN, 4), jnp.float32),
            jax.ShapeDtypeStruct((_B, _TRAIN, 1), jnp.int32),
            jax.ShapeDtypeStruct((_B, _TRAIN, _NUM_CLASSES * 4), jnp.float32),
            jax.ShapeDtypeStruct((_B, _TRAIN, _MASK_H, _MASK_W), jnp.float32),
        ],
        scratch_shapes=[
            pltpu.VMEM((_B * _POS, 4), jnp.float32),
            pltpu.VMEM((_B * _POS, 1), jnp.int32),
            pltpu.VMEM((_B * _POS, _MASK_H, _MASK_IN), jnp.float32),
        ],
    )(props_pad, propsT, gt_boxes, gtT, clsf, masksT)

    cls = cls3[..., 0]
    deltas = deltas2.reshape(_B, _TRAIN, _NUM_CLASSES, 4)
    return rois, cls, deltas, masks


# speedup vs baseline: 1.2081x; 1.0323x over previous
import jax
import jax.numpy as jnp
from jax import lax
from jax.experimental import pallas as pl
from jax.experimental.pallas import tpu as pltpu

_B = 2
_N = 5000
_NP = 5120
_MAX_GT = 100
_POS = 64
_NEG = 192
_TRAIN = 256
_NUM_CLASSES = 81
_MASK_IN = 128
_MASK_H = 28
_MASK_W = 28
_EPS = 1e-6
_INV_SCALE = 1.0 / 512.0
_HI = lax.Precision.HIGHEST


def _to8(row):
    return jnp.concatenate([row[:, 640 * r:640 * (r + 1)] for r in range(8)],
                           axis=0)


def _flat(t8):
    return jnp.concatenate([t8[r:r + 1, :] for r in range(8)], axis=1)


def _roi_kernel(props_ref, propsT_ref, gt_ref, gtT_ref, clsf_ref, masks_hbm,
                rois_ref, cls_ref, deltas_ref, masks_out_ref,
                pr_ref, asg_ref, wy_ref, masks_ref, dma_sem):
    masks_cp = pltpu.make_async_copy(masks_hbm, masks_ref, dma_sem)
    masks_cp.start()
    lane = lax.broadcasted_iota(jnp.int32, (1, _NP), 1)
    idx8 = (lax.broadcasted_iota(jnp.int32, (8, 640), 0) * 640
            + lax.broadcasted_iota(jnp.int32, (8, 640), 1))

    m_pos = []
    m_neg = []
    for b in range(_B):
        gt_n = gt_ref[b] * _INV_SCALE
        pT = propsT_ref[b]
        g0 = gt_n[:, 0:1]
        g1 = gt_n[:, 1:2]
        g2 = gt_n[:, 2:3]
        g3 = gt_n[:, 3:4]
        p0 = pT[0:1, :]
        p1 = pT[1:2, :]
        p2 = pT[2:3, :]
        p3 = pT[3:4, :]
        yy1 = jnp.maximum(g0, p0)
        xx1 = jnp.maximum(g1, p1)
        yy2 = jnp.minimum(g2, p2)
        xx2 = jnp.minimum(g3, p3)
        inter = jnp.maximum(yy2 - yy1, 0.0) * jnp.maximum(xx2 - xx1, 0.0)
        area_p = (p2 - p0) * (p3 - p1)
        area_g = (g2 - g0) * (g3 - g1)
        ovT = inter / (area_p + area_g - inter + _EPS)
        rmax = jnp.max(ovT, axis=0, keepdims=True)
        valid = lane < _N
        m_pos.append(_to8(jnp.where(valid, rmax, -1.0)))
        m_neg.append(_to8(jnp.where(valid, rmax, 2.0)))

    def _extract_max(m, stamp):
        v = jnp.max(jnp.max(m, axis=1, keepdims=True), axis=0,
                    keepdims=True)
        t = jnp.where(m == v, idx8, _NP)
        j = jnp.min(jnp.min(t, axis=1, keepdims=True), axis=0,
                    keepdims=True)
        return jnp.where(idx8 == j, stamp, m)

    def _extract_min(n, stamp):
        w = jnp.min(jnp.min(n, axis=1, keepdims=True), axis=0,
                    keepdims=True)
        t = jnp.where(n == w, idx8, -1)
        k = jnp.max(jnp.max(t, axis=1, keepdims=True), axis=0,
                    keepdims=True)
        return jnp.where(idx8 == k, stamp, n)

    def both_body(i, c):
        m0, m1, n0, n1 = c
        fi = i.astype(jnp.float32)
        pstamp = -3.0 - fi
        nstamp = 258.0 - fi
        m0 = _extract_max(m0, pstamp)
        m1 = _extract_max(m1, pstamp)
        n0 = _extract_min(n0, nstamp)
        n1 = _extract_min(n1, nstamp)
        return m0, m1, n0, n1

    m0, m1, n0, n1 = lax.fori_loop(
        0, _POS, both_body, (m_pos[0], m_pos[1], m_neg[0], m_neg[1]))

    def neg_body(i, c):
        n0, n1 = c
        nstamp = 258.0 - i.astype(jnp.float32)
        n0 = _extract_min(n0, nstamp)
        n1 = _extract_min(n1, nstamp)
        return n0, n1

    n0, n1 = lax.fori_loop(_POS, _NEG, neg_body, (n0, n1))

    rposf = lax.broadcasted_iota(jnp.int32, (_POS, 1), 0).astype(
        jnp.float32) + 1.0
    rnegf = lax.broadcasted_iota(jnp.int32, (_NEG, 1), 0).astype(
        jnp.float32) + (_POS + 1.0)
    i100 = lax.broadcasted_iota(jnp.int32, (_POS, _MAX_GT), 1)
    lane324 = lax.broadcasted_iota(jnp.int32, (_POS, _NUM_CLASSES * 4), 1)

    for b, msel, nsel in ((0, m0, n0), (1, m1, n1)):
        pm_row = -_flat(msel) - 2.0
        nm_row = _flat(nsel) - 2.0
        P = (pm_row == rposf).astype(jnp.float32)
        Nh = (nm_row == rnegf).astype(jnp.float32)
        props = props_ref[b]
        pos_rois = jnp.dot(P, props, preferred_element_type=jnp.float32,
                           precision=_HI)
        neg_rois = jnp.dot(Nh, props, preferred_element_type=jnp.float32,
                           precision=_HI)

        gtT_n = gtT_ref[b] * _INV_SCALE
        q0 = gtT_n[0:1, :]
        q1 = gtT_n[1:2, :]
        q2 = gtT_n[2:3, :]
        q3 = gtT_n[3:4, :]
        r0 = pos_rois[:, 0:1]
        r1 = pos_rois[:, 1:2]
        r2 = pos_rois[:, 2:3]
        r3 = pos_rois[:, 3:4]
        py1 = jnp.maximum(r0, q0)
        px1 = jnp.maximum(r1, q1)
        py2 = jnp.minimum(r2, q2)
        px2 = jnp.minimum(r3, q3)
        pinter = jnp.maximum(py2 - py1, 0.0) * jnp.maximum(px2 - px1, 0.0)
        par = (r2 - r0) * (r3 - r1)
        pag = (q2 - q0) * (q3 - q1)
        po = pinter / (par + pag - pinter + _EPS)

        amax = jnp.max(po, axis=1, keepdims=True)
        asg = jnp.min(jnp.where(po == amax, i100, _MAX_GT), axis=1,
                      keepdims=True)
        A = (i100 == asg).astype(jnp.float32)
        gt_n = gt_ref[b] * _INV_SCALE
        roi_gt = jnp.dot(A, gt_n, preferred_element_type=jnp.float32,
                         precision=_HI)
        idsf = jnp.dot(A, clsf_ref[b], preferred_element_type=jnp.float32,
                       precision=_HI)
        ids = (idsf + 0.5).astype(jnp.int32)

        h = pos_rois[:, 2:3] - pos_rois[:, 0:1] + _EPS
        w = pos_rois[:, 3:4] - pos_rois[:, 1:2] + _EPS
        cy = pos_rois[:, 0:1] + 0.5 * h
        cx = pos_rois[:, 1:2] + 0.5 * w
        gh = roi_gt[:, 2:3] - roi_gt[:, 0:1] + _EPS
        gw = roi_gt[:, 3:4] - roi_gt[:, 1:2] + _EPS
        gcy = roi_gt[:, 0:1] + 0.5 * gh
        gcx = roi_gt[:, 1:2] + 0.5 * gw
        dy = ((gcy - cy) / h) / 0.1
        dx = ((gcx - cx) / w) / 0.1
        dh = jnp.log(gh / h) / 0.2
        dw = jnp.log(gw / w) / 0.2

        cls_l = lane324 // 4
        d_l = lane324 % 4
        dval = jnp.where(d_l == 0, dy,
                         jnp.where(d_l == 1, dx,
                                   jnp.where(d_l == 2, dh, dw)))
        cd = jnp.where(cls_l == ids, dval, 0.0)

        lin28c = lax.broadcasted_iota(jnp.int32, (_POS, _MASK_H), 1).astype(
            jnp.float32) * (1.0 / (_MASK_H - 1))
        iota3 = lax.broadcasted_iota(jnp.int32,
                                     (_POS, _MASK_H, _MASK_IN), 2)
        ys = jnp.clip((r0 + (r2 - r0) * lin28c) * (_MASK_IN - 1.0),
                      0.0, _MASK_IN - 1.0)
        y0f = jnp.floor(ys)
        wy = (ys - y0f)[:, :, None]
        y0i = y0f.astype(jnp.int32)[:, :, None]
        y1i = jnp.minimum(y0i + 1, _MASK_IN - 1)
        wy_ref[pl.ds(_POS * b, _POS)] = (
            jnp.where(iota3 == y0i, 1.0 - wy, 0.0)
            + jnp.where(iota3 == y1i, wy, 0.0))
        pr_ref[pl.ds(_POS * b, _POS), :] = pos_rois
        asg_ref[pl.ds(_POS * b, _POS), :] = asg
        rois_ref[b, pl.ds(0, _POS), :] = pos_rois
        rois_ref[b, pl.ds(_POS, _NEG), :] = neg_rois
        cls_ref[b, pl.ds(0, _POS), :] = ids
        cls_ref[b, pl.ds(_POS, _NEG), :] = jnp.zeros((_NEG, 1), jnp.int32)
        deltas_ref[b, pl.ds(0, _POS), :] = cd
        deltas_ref[b, pl.ds(_POS, _NEG), :] = jnp.zeros(
            (_NEG, _NUM_CLASSES * 4), jnp.float32)
        masks_out_ref[b, pl.ds(_POS, _NEG)] = jnp.zeros(
            (_NEG, _MASK_H, _MASK_W), jnp.float32)

    lin_row = lax.broadcasted_iota(jnp.int32, (1, _MASK_W), 1).astype(
        jnp.float32) * (1.0 / (_MASK_W - 1))
    xj = lax.broadcasted_iota(jnp.int32, (_MASK_IN, _MASK_W), 0)

    masks_cp.wait()

    def mask_body(k, carry):
        for u in range(8):
            r = k + 16 * u
            b = u // 4
            rr = k + 16 * (u % 4)
            row = pr_ref[pl.ds(r, 1), :]
            b1 = row[:, 1:2]
            b3 = row[:, 3:4]
            ch = asg_ref[r, 0]
            M = masks_ref[b, ch]
            Wy = wy_ref[r]

            xs = jnp.clip((b1 + (b3 - b1) * lin_row) * (_MASK_IN - 1.0),
                          0.0, _MASK_IN - 1.0)
            x0f = jnp.floor(xs)
            wx = xs - x0f
            x0i = x0f.astype(jnp.int32)
            x1i = jnp.minimum(x0i + 1, _MASK_IN - 1)
            WxT = (jnp.where(xj == x0i, 1.0 - wx, 0.0)
                   + jnp.where(xj == x1i, wx, 0.0))

            tmp = jnp.dot(Wy, M, preferred_element_type=jnp.float32,
                          precision=_HI)
            out = jnp.dot(tmp, WxT, preferred_element_type=jnp.float32,
                          precision=_HI)
            masks_out_ref[b, rr] = out
        return carry

    lax.fori_loop(0, 16, mask_body, 0)


def kernel(proposals, gt_class_ids, gt_boxes, gt_masks):
    props_pad = jnp.pad(proposals, ((0, 0), (0, _NP - _N), (0, 0)))
    propsT = jnp.transpose(props_pad, (0, 2, 1))
    gtT = jnp.transpose(gt_boxes, (0, 2, 1))
    clsf = gt_class_ids.astype(jnp.float32)[..., None]
    masksT = jnp.transpose(gt_masks, (0, 3, 1, 2))

    rois, cls3, deltas2, masks = pl.pallas_call(
        _roi_kernel,
        out_shape=[
            jax.ShapeDtypeStruct((_B, _TRAIN, 4), jnp.float32),
            jax.ShapeDtypeStruct((_B, _TRAIN, 1), jnp.int32),
            jax.ShapeDtypeStruct((_B, _TRAIN, _NUM_CLASSES * 4), jnp.float32),
            jax.ShapeDtypeStruct((_B, _TRAIN, _MASK_H, _MASK_W), jnp.float32),
        ],
        in_specs=[
            pl.BlockSpec(memory_space=pl.ANY) if i == 5
            else pl.BlockSpec()
            for i in range(6)
        ],
        scratch_shapes=[
            pltpu.VMEM((_B * _POS, 4), jnp.float32),
            pltpu.VMEM((_B * _POS, 1), jnp.int32),
            pltpu.VMEM((_B * _POS, _MASK_H, _MASK_IN), jnp.float32),
            pltpu.VMEM((_B, _MAX_GT, _MASK_IN, _MASK_IN), jnp.float32),
            pltpu.SemaphoreType.DMA,
        ],
    )(props_pad, propsT, gt_boxes, gtT, clsf, masksT)

    cls = cls3[..., 0]
    deltas = deltas2.reshape(_B, _TRAIN, _NUM_CLASSES, 4)
    return rois, cls, deltas, masks


# speedup vs baseline: 1.2136x; 1.0045x over previous
import jax
import jax.numpy as jnp
from jax import lax
from jax.experimental import pallas as pl
from jax.experimental.pallas import tpu as pltpu

_B = 2
_N = 5000
_NP = 5120
_MAX_GT = 100
_POS = 64
_NEG = 192
_TRAIN = 256
_NUM_CLASSES = 81
_MASK_IN = 128
_MASK_H = 28
_MASK_W = 28
_EPS = 1e-6
_INV_SCALE = 1.0 / 512.0
_HI = lax.Precision.HIGHEST


def _to8(row):
    return jnp.concatenate([row[:, 640 * r:640 * (r + 1)] for r in range(8)],
                           axis=0)


def _flat(t8):
    return jnp.concatenate([t8[r:r + 1, :] for r in range(8)], axis=1)


def _roi_kernel(props_ref, propsT_ref, gt_ref, gtT_ref, clsf_ref, masks_hbm,
                rois_ref, cls_ref, deltas_ref, masks_out_ref,
                pr_ref, asg_ref, wy_ref, masks_ref, dma_sem):
    masks_cp = pltpu.make_async_copy(masks_hbm, masks_ref, dma_sem)
    masks_cp.start()
    lane = lax.broadcasted_iota(jnp.int32, (1, _NP), 1)
    idx8 = (lax.broadcasted_iota(jnp.int32, (8, 640), 0) * 640
            + lax.broadcasted_iota(jnp.int32, (8, 640), 1))

    m_pos = []
    m_neg = []
    for b in range(_B):
        gt_n = gt_ref[b] * _INV_SCALE
        pT = propsT_ref[b]
        g0 = gt_n[:, 0:1]
        g1 = gt_n[:, 1:2]
        g2 = gt_n[:, 2:3]
        g3 = gt_n[:, 3:4]
        p0 = pT[0:1, :]
        p1 = pT[1:2, :]
        p2 = pT[2:3, :]
        p3 = pT[3:4, :]
        yy1 = jnp.maximum(g0, p0)
        xx1 = jnp.maximum(g1, p1)
        yy2 = jnp.minimum(g2, p2)
        xx2 = jnp.minimum(g3, p3)
        inter = jnp.maximum(yy2 - yy1, 0.0) * jnp.maximum(xx2 - xx1, 0.0)
        area_p = (p2 - p0) * (p3 - p1)
        area_g = (g2 - g0) * (g3 - g1)
        ovT = inter / (area_p + area_g - inter + _EPS)
        rmax = jnp.max(ovT, axis=0, keepdims=True)
        valid = lane < _N
        m_pos.append(_to8(jnp.where(valid, rmax, -1.0)))
        m_neg.append(_to8(jnp.where(valid, rmax, 2.0)))

    def _extract_max(m, stamp):
        v = jnp.max(jnp.max(m, axis=1, keepdims=True), axis=0,
                    keepdims=True)
        t = jnp.where(m == v, idx8, _NP)
        j = jnp.min(jnp.min(t, axis=1, keepdims=True), axis=0,
                    keepdims=True)
        return jnp.where(idx8 == j, stamp, m)

    def _extract_min(n, stamp):
        w = jnp.min(jnp.min(n, axis=1, keepdims=True), axis=0,
                    keepdims=True)
        t = jnp.where(n == w, idx8, -1)
        k = jnp.max(jnp.max(t, axis=1, keepdims=True), axis=0,
                    keepdims=True)
        return jnp.where(idx8 == k, stamp, n)

    def both_body(i, c):
        m0, m1, n0, n1 = c
        fi = (2 * i).astype(jnp.float32)
        pstamp = -3.0 - fi
        nstamp = 258.0 - fi
        m0 = _extract_max(m0, pstamp)
        m1 = _extract_max(m1, pstamp)
        n0 = _extract_min(n0, nstamp)
        n1 = _extract_min(n1, nstamp)
        m0 = _extract_max(m0, pstamp - 1.0)
        m1 = _extract_max(m1, pstamp - 1.0)
        n0 = _extract_min(n0, nstamp - 1.0)
        n1 = _extract_min(n1, nstamp - 1.0)
        return m0, m1, n0, n1

    m0, m1, n0, n1 = lax.fori_loop(
        0, _POS // 2, both_body, (m_pos[0], m_pos[1], m_neg[0], m_neg[1]))

    def neg_body(i, c):
        n0, n1 = c
        nstamp = 258.0 - (_POS + 2 * i).astype(jnp.float32)
        n0 = _extract_min(n0, nstamp)
        n1 = _extract_min(n1, nstamp)
        n0 = _extract_min(n0, nstamp - 1.0)
        n1 = _extract_min(n1, nstamp - 1.0)
        return n0, n1

    n0, n1 = lax.fori_loop(0, (_NEG - _POS) // 2, neg_body, (n0, n1))

    rposf = lax.broadcasted_iota(jnp.int32, (_POS, 1), 0).astype(
        jnp.float32) + 1.0
    rnegf = lax.broadcasted_iota(jnp.int32, (_NEG, 1), 0).astype(
        jnp.float32) + (_POS + 1.0)
    i100 = lax.broadcasted_iota(jnp.int32, (_POS, _MAX_GT), 1)
    lane324 = lax.broadcasted_iota(jnp.int32, (_POS, _NUM_CLASSES * 4), 1)

    for b, msel, nsel in ((0, m0, n0), (1, m1, n1)):
        pm_row = -_flat(msel) - 2.0
        nm_row = _flat(nsel) - 2.0
        P = (pm_row == rposf).astype(jnp.float32)
        Nh = (nm_row == rnegf).astype(jnp.float32)
        props = props_ref[b]
        pos_rois = jnp.dot(P, props, preferred_element_type=jnp.float32,
                           precision=_HI)
        neg_rois = jnp.dot(Nh, props, preferred_element_type=jnp.float32,
                           precision=_HI)

        gtT_n = gtT_ref[b] * _INV_SCALE
        q0 = gtT_n[0:1, :]
        q1 = gtT_n[1:2, :]
        q2 = gtT_n[2:3, :]
        q3 = gtT_n[3:4, :]
        r0 = pos_rois[:, 0:1]
        r1 = pos_rois[:, 1:2]
        r2 = pos_rois[:, 2:3]
        r3 = pos_rois[:, 3:4]
        py1 = jnp.maximum(r0, q0)
        px1 = jnp.maximum(r1, q1)
        py2 = jnp.minimum(r2, q2)
        px2 = jnp.minimum(r3, q3)
        pinter = jnp.maximum(py2 - py1, 0.0) * jnp.maximum(px2 - px1, 0.0)
        par = (r2 - r0) * (r3 - r1)
        pag = (q2 - q0) * (q3 - q1)
        po = pinter / (par + pag - pinter + _EPS)

        amax = jnp.max(po, axis=1, keepdims=True)
        asg = jnp.min(jnp.where(po == amax, i100, _MAX_GT), axis=1,
                      keepdims=True)
        A = (i100 == asg).astype(jnp.float32)
        gt_n = gt_ref[b] * _INV_SCALE
        roi_gt = jnp.dot(A, gt_n, preferred_element_type=jnp.float32,
                         precision=_HI)
        idsf = jnp.dot(A, clsf_ref[b], preferred_element_type=jnp.float32,
                       precision=_HI)
        ids = (idsf + 0.5).astype(jnp.int32)

        h = pos_rois[:, 2:3] - pos_rois[:, 0:1] + _EPS
        w = pos_rois[:, 3:4] - pos_rois[:, 1:2] + _EPS
        cy = pos_rois[:, 0:1] + 0.5 * h
        cx = pos_rois[:, 1:2] + 0.5 * w
        gh = roi_gt[:, 2:3] - roi_gt[:, 0:1] + _EPS
        gw = roi_gt[:, 3:4] - roi_gt[:, 1:2] + _EPS
        gcy = roi_gt[:, 0:1] + 0.5 * gh
        gcx = roi_gt[:, 1:2] + 0.5 * gw
        dy = ((gcy - cy) / h) / 0.1
        dx = ((gcx - cx) / w) / 0.1
        dh = jnp.log(gh / h) / 0.2
        dw = jnp.log(gw / w) / 0.2

        cls_l = lane324 // 4
        d_l = lane324 % 4
        dval = jnp.where(d_l == 0, dy,
                         jnp.where(d_l == 1, dx,
                                   jnp.where(d_l == 2, dh, dw)))
        cd = jnp.where(cls_l == ids, dval, 0.0)

        lin28c = lax.broadcasted_iota(jnp.int32, (_POS, _MASK_H), 1).astype(
            jnp.float32) * (1.0 / (_MASK_H - 1))
        iota3 = lax.broadcasted_iota(jnp.int32,
                                     (_POS, _MASK_H, _MASK_IN), 2)
        ys = jnp.clip((r0 + (r2 - r0) * lin28c) * (_MASK_IN - 1.0),
                      0.0, _MASK_IN - 1.0)
        y0f = jnp.floor(ys)
        wy = (ys - y0f)[:, :, None]
        y0i = y0f.astype(jnp.int32)[:, :, None]
        y1i = jnp.minimum(y0i + 1, _MASK_IN - 1)
        wy_ref[pl.ds(_POS * b, _POS)] = (
            jnp.where(iota3 == y0i, 1.0 - wy, 0.0)
            + jnp.where(iota3 == y1i, wy, 0.0))
        pr_ref[pl.ds(_POS * b, _POS), :] = pos_rois
        asg_ref[pl.ds(_POS * b, _POS), :] = asg
        rois_ref[b, pl.ds(0, _POS), :] = pos_rois
        rois_ref[b, pl.ds(_POS, _NEG), :] = neg_rois
        cls_ref[b, pl.ds(0, _POS), :] = ids
        cls_ref[b, pl.ds(_POS, _NEG), :] = jnp.zeros((_NEG, 1), jnp.int32)
        deltas_ref[b, pl.ds(0, _POS), :] = cd
        deltas_ref[b, pl.ds(_POS, _NEG), :] = jnp.zeros(
            (_NEG, _NUM_CLASSES * 4), jnp.float32)
        masks_out_ref[b, pl.ds(_POS, _NEG)] = jnp.zeros(
            (_NEG, _MASK_H, _MASK_W), jnp.float32)

    lin_row = lax.broadcasted_iota(jnp.int32, (1, _MASK_W), 1).astype(
        jnp.float32) * (1.0 / (_MASK_W - 1))
    xj = lax.broadcasted_iota(jnp.int32, (_MASK_IN, _MASK_W), 0)

    masks_cp.wait()

    def mask_body(k, carry):
        for u in range(8):
            r = k + 16 * u
            b = u // 4
            rr = k + 16 * (u % 4)
            row = pr_ref[pl.ds(r, 1), :]
            b1 = row[:, 1:2]
            b3 = row[:, 3:4]
            ch = asg_ref[r, 0]
            M = masks_ref[b, ch]
            Wy = wy_ref[r]

            xs = jnp.clip((b1 + (b3 - b1) * lin_row) * (_MASK_IN - 1.0),
                          0.0, _MASK_IN - 1.0)
            x0f = jnp.floor(xs)
            wx = xs - x0f
            x0i = x0f.astype(jnp.int32)
            x1i = jnp.minimum(x0i + 1, _MASK_IN - 1)
            WxT = (jnp.where(xj == x0i, 1.0 - wx, 0.0)
                   + jnp.where(xj == x1i, wx, 0.0))

            tmp = jnp.dot(Wy, M, preferred_element_type=jnp.float32,
                          precision=_HI)
            out = jnp.dot(tmp, WxT, preferred_element_type=jnp.float32,
                          precision=_HI)
            masks_out_ref[b, rr] = out
        return carry

    lax.fori_loop(0, 16, mask_body, 0)


def kernel(proposals, gt_class_ids, gt_boxes, gt_masks):
    props_pad = jnp.pad(proposals, ((0, 0), (0, _NP - _N), (0, 0)))
    propsT = jnp.transpose(props_pad, (0, 2, 1))
    gtT = jnp.transpose(gt_boxes, (0, 2, 1))
    clsf = gt_class_ids.astype(jnp.float32)[..., None]
    masksT = jnp.transpose(gt_masks, (0, 3, 1, 2))

    rois, cls3, deltas2, masks = pl.pallas_call(
        _roi_kernel,
        out_shape=[
            jax.ShapeDtypeStruct((_B, _TRAIN, 4), jnp.float32),
            jax.ShapeDtypeStruct((_B, _TRAIN, 1), jnp.int32),
            jax.ShapeDtypeStruct((_B, _TRAIN, _NUM_CLASSES * 4), jnp.float32),
            jax.ShapeDtypeStruct((_B, _TRAIN, _MASK_H, _MASK_W), jnp.float32),
        ],
        in_specs=[
            pl.BlockSpec(memory_space=pl.ANY) if i == 5
            else pl.BlockSpec()
            for i in range(6)
        ],
        scratch_shapes=[
            pltpu.VMEM((_B * _POS, 4), jnp.float32),
            pltpu.VMEM((_B * _POS, 1), jnp.int32),
            pltpu.VMEM((_B * _POS, _MASK_H, _MASK_IN), jnp.float32),
            pltpu.VMEM((_B, _MAX_GT, _MASK_IN, _MASK_IN), jnp.float32),
            pltpu.SemaphoreType.DMA,
        ],
    )(props_pad, propsT, gt_boxes, gtT, clsf, masksT)

    cls = cls3[..., 0]
    deltas = deltas2.reshape(_B, _TRAIN, _NUM_CLASSES, 4)
    return rois, cls, deltas, masks
